# baseline (speedup 1.0000x reference)
import jax
import jax.numpy as jnp
from jax import lax
from jax.experimental import pallas as pl
from jax.experimental.pallas import tpu as pltpu
from jax.experimental.pallas import tpu_sc as plsc

NUM_ROWS = 1000000
NGRAN = NUM_ROWS // 16
BATCH = 16384
DIM = 64
WINDOW = 32
LANES = 16
IDXLEN = DIM * WINDOW


def kernel(user, item, user_table, item_table):
    ut4 = user_table.T.reshape(DIM * NGRAN, 16)
    it4 = item_table.T.reshape(DIM * NGRAN, 16)
    u2 = user.reshape(1, BATCH)
    i2 = item.reshape(1, BATCH)

    mesh = plsc.VectorSubcoreMesh(core_axis_name="core",
                                  subcore_axis_name="subcore")

    @pl.kernel(
        out_type=jax.ShapeDtypeStruct((DIM, BATCH), jnp.float32),
        mesh=mesh,
        compiler_params=pltpu.CompilerParams(use_tc_tiling_on_sc=False,
                                             needs_layout_passes=False),
        scratch_types=[
            pltpu.VMEM((IDXLEN, 16), jnp.float32),
            pltpu.VMEM((IDXLEN, 16), jnp.float32),
            pltpu.VMEM((IDXLEN,), jnp.int32),
            pltpu.VMEM((IDXLEN,), jnp.int32),
            pltpu.VMEM((WINDOW,), jnp.int32),
            pltpu.VMEM((WINDOW,), jnp.int32),
            pltpu.SemaphoreType.DMA,
            pltpu.SemaphoreType.DMA,
        ],
    )
    def sc_kernel(u_hbm, i_hbm, ut_hbm, it_hbm, o_hbm,
                  ug, ig, uidx, iidx, ulan, ilan, sem_u, sem_i):
        def body(u_idx, i_idx, o_vmem):
            @pl.loop(0, WINDOW, step=LANES)
            def _(c):
                uv = u_idx.at[0][pl.ds(c, LANES)]
                iv = i_idx.at[0][pl.ds(c, LANES)]
                uidx.at[pl.ds(c, LANES)][...] = lax.shift_right_logical(uv, 4)
                iidx.at[pl.ds(c, LANES)][...] = lax.shift_right_logical(iv, 4)
                ulan.at[pl.ds(c, LANES)][...] = lax.bitwise_and(uv, 15)
                ilan.at[pl.ds(c, LANES)][...] = lax.bitwise_and(iv, 15)

            @pl.loop(1, DIM)
            def _(d):
                @pl.loop(0, WINDOW, step=LANES)
                def _(c):
                    base = uidx.at[pl.ds(c, LANES)][...]
                    uidx.at[pl.ds(d * WINDOW + c, LANES)][...] = base + d * NGRAN
                    ibase = iidx.at[pl.ds(c, LANES)][...]
                    iidx.at[pl.ds(d * WINDOW + c, LANES)][...] = ibase + d * NGRAN

            cp_u = pltpu.make_async_copy(ut_hbm.at[uidx], ug, sem_u)
            cp_i = pltpu.make_async_copy(it_hbm.at[iidx], ig, sem_i)
            cp_u.start()
            cp_i.start()
            cp_u.wait()
            cp_i.wait()

            @pl.loop(0, DIM)
            def _(d):
                @pl.loop(0, WINDOW, step=LANES)
                def _(c):
                    rows = lax.iota(jnp.int32, LANES) + (d * WINDOW + c)
                    uvals = plsc.load_gather(
                        ug, [rows, ulan.at[pl.ds(c, LANES)][...]])
                    ivals = plsc.load_gather(
                        ig, [rows, ilan.at[pl.ds(c, LANES)][...]])
                    o_vmem.at[d][pl.ds(c, LANES)] = uvals * ivals

        pltpu.emit_pipeline(
            body,
            grid=(BATCH // WINDOW,),
            in_specs=[
                pl.BlockSpec((1, WINDOW), lambda i: (0, i)),
                pl.BlockSpec((1, WINDOW), lambda i: (0, i)),
            ],
            out_specs=[pl.BlockSpec((DIM, WINDOW), lambda i: (0, i))],
            core_axis_name=("core", "subcore"),
            dimension_semantics=(pltpu.PARALLEL,),
        )(u_hbm, i_hbm, o_hbm)

    out = sc_kernel(u2, i2, ut4, it4)
    return out.T


# speedup vs baseline: 4.0662x; 4.0662x over previous
import jax
import jax.numpy as jnp
from jax import lax
from jax.experimental import pallas as pl
from jax.experimental.pallas import tpu as pltpu
from jax.experimental.pallas import tpu_sc as plsc

NUM_ROWS = 1000000
NBLK = 1954
NPROW = NBLK * 256
BATCH = 16384
DIM = 64
TCC = 512
WINDOW = 128
LANES = 16


def _tc_transpose(tT):
    def body(in_ref, out_ref):
        t = jnp.transpose(in_ref[...])
        out_ref[...] = jnp.concatenate([t[:TCC // 2], t[TCC // 2:]], axis=1)

    return pl.pallas_call(
        body,
        grid=(pl.cdiv(NUM_ROWS, TCC),),
        in_specs=[pl.BlockSpec((DIM, TCC), lambda i: (0, i))],
        out_specs=pl.BlockSpec((TCC // 2, 2 * DIM), lambda i: (i, 0)),
        out_shape=jax.ShapeDtypeStruct((NPROW, 2 * DIM), jnp.float32),
        compiler_params=pltpu.CompilerParams(
            dimension_semantics=("parallel",)),
    )(tT)


def kernel(user, item, user_table, item_table):
    ut2 = _tc_transpose(user_table.T)
    it2 = _tc_transpose(item_table.T)
    u2 = user.reshape(1, BATCH)
    i2 = item.reshape(1, BATCH)

    mesh = plsc.VectorSubcoreMesh(core_axis_name="core",
                                  subcore_axis_name="subcore")

    @pl.kernel(
        out_type=jax.ShapeDtypeStruct((DIM, BATCH), jnp.float32),
        mesh=mesh,
        compiler_params=pltpu.CompilerParams(use_tc_tiling_on_sc=False,
                                             needs_layout_passes=False),
        scratch_types=[
            pltpu.VMEM((WINDOW, 2 * DIM), jnp.float32),
            pltpu.VMEM((WINDOW, 2 * DIM), jnp.float32),
            pltpu.VMEM((WINDOW,), jnp.int32),
            pltpu.VMEM((WINDOW,), jnp.int32),
            pltpu.VMEM((WINDOW,), jnp.int32),
            pltpu.VMEM((WINDOW,), jnp.int32),
            pltpu.SemaphoreType.DMA,
            pltpu.SemaphoreType.DMA,
        ],
    )
    def sc_kernel(u_hbm, i_hbm, ut_hbm, it_hbm, o_hbm,
                  ug, ig, up, ip, uh, ih, sem_u, sem_i):
        def body(u_idx, i_idx, o_vmem):
            @pl.loop(0, WINDOW, step=LANES)
            def _(c):
                uv = u_idx.at[0][pl.ds(c, LANES)]
                iv = i_idx.at[0][pl.ds(c, LANES)]
                up.at[pl.ds(c, LANES)][...] = (
                    lax.shift_left(lax.shift_right_logical(uv, 9), 8)
                    + lax.bitwise_and(uv, 255))
                ip.at[pl.ds(c, LANES)][...] = (
                    lax.shift_left(lax.shift_right_logical(iv, 9), 8)
                    + lax.bitwise_and(iv, 255))
                uh.at[pl.ds(c, LANES)][...] = (
                    lax.bitwise_and(lax.shift_right_logical(uv, 8), 1) * DIM)
                ih.at[pl.ds(c, LANES)][...] = (
                    lax.bitwise_and(lax.shift_right_logical(iv, 8), 1) * DIM)

            cp_u = pltpu.make_async_copy(ut_hbm.at[up], ug, sem_u)
            cp_i = pltpu.make_async_copy(it_hbm.at[ip], ig, sem_i)
            cp_u.start()
            cp_i.start()
            cp_u.wait()
            cp_i.wait()

            @pl.loop(0, DIM)
            def _(d):
                @pl.loop(0, WINDOW, step=LANES)
                def _(c):
                    rows = lax.iota(jnp.int32, LANES) + c
                    ucols = uh.at[pl.ds(c, LANES)][...] + d
                    icols = ih.at[pl.ds(c, LANES)][...] + d
                    uvals = plsc.load_gather(ug, [rows, ucols])
                    ivals = plsc.load_gather(ig, [rows, icols])
                    o_vmem.at[d][pl.ds(c, LANES)] = uvals * ivals

        pltpu.emit_pipeline(
            body,
            grid=(BATCH // WINDOW,),
            in_specs=[
                pl.BlockSpec((1, WINDOW), lambda i: (0, i)),
                pl.BlockSpec((1, WINDOW), lambda i: (0, i)),
            ],
            out_specs=[pl.BlockSpec((DIM, WINDOW), lambda i: (0, i))],
            core_axis_name=("core", "subcore"),
            dimension_semantics=(pltpu.PARALLEL,),
        )(u_hbm, i_hbm, o_hbm)

    out = sc_kernel(u2, i2, ut2, it2)
    return out.T


# speedup vs baseline: 7.8917x; 1.9408x over previous
import jax
import jax.numpy as jnp
from jax import lax
from jax.experimental import pallas as pl
from jax.experimental.pallas import tpu as pltpu
from jax.experimental.pallas import tpu_sc as plsc

NUM_ROWS = 1000000
NBLK = 489
NPROW = NBLK * 1024
BATCH = 16384
DIM = 64
TCC = 2048
WINDOW = 128
LANES = 16


def _tc_transpose(tT):
    def body(in_ref, out_ref):
        row = lax.broadcasted_iota(jnp.int32, (DIM, DIM), 0)
        col = lax.broadcasted_iota(jnp.int32, (DIM, DIM), 1)
        ident = jnp.where(row == col, 1.0, 0.0).astype(jnp.float32)
        t = lax.dot_general(in_ref[...], ident, (((0,), (0,)), ((), ())),
                            precision=lax.Precision.HIGHEST)
        out_ref[...] = jnp.concatenate([t[:TCC // 2], t[TCC // 2:]], axis=1)

    return pl.pallas_call(
        body,
        grid=(pl.cdiv(NUM_ROWS, TCC),),
        in_specs=[pl.BlockSpec((DIM, TCC), lambda i: (0, i))],
        out_specs=pl.BlockSpec((TCC // 2, 2 * DIM), lambda i: (i, 0)),
        out_shape=jax.ShapeDtypeStruct((NPROW, 2 * DIM), jnp.float32),
        compiler_params=pltpu.CompilerParams(
            dimension_semantics=("parallel",)),
    )(tT)


def kernel(user, item, user_table, item_table):
    ut2 = _tc_transpose(user_table.T)
    it2 = _tc_transpose(item_table.T)
    u2 = user.reshape(1, BATCH)
    i2 = item.reshape(1, BATCH)

    mesh = plsc.VectorSubcoreMesh(core_axis_name="core",
                                  subcore_axis_name="subcore")

    @pl.kernel(
        out_type=jax.ShapeDtypeStruct((DIM, BATCH), jnp.float32),
        mesh=mesh,
        compiler_params=pltpu.CompilerParams(use_tc_tiling_on_sc=False,
                                             needs_layout_passes=False),
        scratch_types=[
            pltpu.VMEM((WINDOW, 2 * DIM), jnp.float32),
            pltpu.VMEM((WINDOW, 2 * DIM), jnp.float32),
            pltpu.VMEM((WINDOW,), jnp.int32),
            pltpu.VMEM((WINDOW,), jnp.int32),
            pltpu.VMEM((WINDOW,), jnp.int32),
            pltpu.VMEM((WINDOW,), jnp.int32),
            pltpu.SemaphoreType.DMA,
            pltpu.SemaphoreType.DMA,
        ],
    )
    def sc_kernel(u_hbm, i_hbm, ut_hbm, it_hbm, o_hbm,
                  ug, ig, up, ip, uh, ih, sem_u, sem_i):
        def body(u_idx, i_idx, o_vmem):
            @pl.loop(0, WINDOW, step=LANES)
            def _(c):
                uv = u_idx.at[0][pl.ds(c, LANES)]
                iv = i_idx.at[0][pl.ds(c, LANES)]
                up.at[pl.ds(c, LANES)][...] = (
                    lax.shift_left(lax.shift_right_logical(uv, 11), 10)
                    + lax.bitwise_and(uv, 1023))
                ip.at[pl.ds(c, LANES)][...] = (
                    lax.shift_left(lax.shift_right_logical(iv, 11), 10)
                    + lax.bitwise_and(iv, 1023))
                uh.at[pl.ds(c, LANES)][...] = (
                    lax.bitwise_and(lax.shift_right_logical(uv, 10), 1) * DIM)
                ih.at[pl.ds(c, LANES)][...] = (
                    lax.bitwise_and(lax.shift_right_logical(iv, 10), 1) * DIM)

            cp_u = pltpu.make_async_copy(ut_hbm.at[up], ug, sem_u)
            cp_i = pltpu.make_async_copy(it_hbm.at[ip], ig, sem_i)
            cp_u.start()
            cp_i.start()
            cp_u.wait()
            cp_i.wait()

            @pl.loop(0, DIM)
            def _(d):
                @pl.loop(0, WINDOW, step=LANES)
                def _(c):
                    rows = lax.iota(jnp.int32, LANES) + c
                    ucols = uh.at[pl.ds(c, LANES)][...] + d
                    icols = ih.at[pl.ds(c, LANES)][...] + d
                    uvals = plsc.load_gather(ug, [rows, ucols])
                    ivals = plsc.load_gather(ig, [rows, icols])
                    o_vmem.at[d][pl.ds(c, LANES)] = uvals * ivals

        pltpu.emit_pipeline(
            body,
            grid=(BATCH // WINDOW,),
            in_specs=[
                pl.BlockSpec((1, WINDOW), lambda i: (0, i)),
                pl.BlockSpec((1, WINDOW), lambda i: (0, i)),
            ],
            out_specs=[pl.BlockSpec((DIM, WINDOW), lambda i: (0, i))],
            core_axis_name=("core", "subcore"),
            dimension_semantics=(pltpu.PARALLEL,),
        )(u_hbm, i_hbm, o_hbm)

    out = sc_kernel(u2, i2, ut2, it2)
    return out.T


# speedup vs baseline: 10.0631x; 1.2752x over previous
import jax
import jax.numpy as jnp
from jax import lax
from jax.experimental import pallas as pl
from jax.experimental.pallas import tpu as pltpu
from jax.experimental.pallas import tpu_sc as plsc

NUM_ROWS = 1000000
NBLK = 489
NPROW = NBLK * 1024
BATCH = 16384
DIM = 64
TCC = 2048
WINDOW = 128
LANES = 16


def _tc_transpose(tT):
    def body(in_ref, out_ref):
        row = lax.broadcasted_iota(jnp.int32, (DIM, DIM), 0)
        col = lax.broadcasted_iota(jnp.int32, (DIM, DIM), 1)
        ident = jnp.where(row == col, 1.0, 0.0).astype(jnp.float32)
        t = lax.dot_general(in_ref[...], ident, (((0,), (0,)), ((), ())),
                            precision=lax.Precision.DEFAULT)
        out_ref[...] = jnp.concatenate([t[:TCC // 2], t[TCC // 2:]], axis=1)

    return pl.pallas_call(
        body,
        grid=(pl.cdiv(NUM_ROWS, TCC),),
        in_specs=[pl.BlockSpec((DIM, TCC), lambda i: (0, i))],
        out_specs=pl.BlockSpec((TCC // 2, 2 * DIM), lambda i: (i, 0)),
        out_shape=jax.ShapeDtypeStruct((NPROW, 2 * DIM), jnp.float32),
        compiler_params=pltpu.CompilerParams(
            dimension_semantics=("parallel",)),
    )(tT)


def kernel(user, item, user_table, item_table):
    ut2 = _tc_transpose(user_table.T)
    it2 = _tc_transpose(item_table.T)
    u2 = user.reshape(1, BATCH)
    i2 = item.reshape(1, BATCH)

    mesh = plsc.VectorSubcoreMesh(core_axis_name="core",
                                  subcore_axis_name="subcore")

    @pl.kernel(
        out_type=jax.ShapeDtypeStruct((DIM, BATCH), jnp.float32),
        mesh=mesh,
        compiler_params=pltpu.CompilerParams(use_tc_tiling_on_sc=False,
                                             needs_layout_passes=False),
        scratch_types=[
            pltpu.VMEM((WINDOW, 2 * DIM), jnp.float32),
            pltpu.VMEM((WINDOW, 2 * DIM), jnp.float32),
            pltpu.VMEM((WINDOW,), jnp.int32),
            pltpu.VMEM((WINDOW,), jnp.int32),
            pltpu.VMEM((WINDOW,), jnp.int32),
            pltpu.VMEM((WINDOW,), jnp.int32),
            pltpu.SemaphoreType.DMA,
            pltpu.SemaphoreType.DMA,
        ],
    )
    def sc_kernel(u_hbm, i_hbm, ut_hbm, it_hbm, o_hbm,
                  ug, ig, up, ip, uh, ih, sem_u, sem_i):
        def body(u_idx, i_idx, o_vmem):
            @pl.loop(0, WINDOW, step=LANES)
            def _(c):
                uv = u_idx.at[0][pl.ds(c, LANES)]
                iv = i_idx.at[0][pl.ds(c, LANES)]
                up.at[pl.ds(c, LANES)][...] = (
                    lax.shift_left(lax.shift_right_logical(uv, 11), 10)
                    + lax.bitwise_and(uv, 1023))
                ip.at[pl.ds(c, LANES)][...] = (
                    lax.shift_left(lax.shift_right_logical(iv, 11), 10)
                    + lax.bitwise_and(iv, 1023))
                uh.at[pl.ds(c, LANES)][...] = (
                    lax.bitwise_and(lax.shift_right_logical(uv, 10), 1) * DIM)
                ih.at[pl.ds(c, LANES)][...] = (
                    lax.bitwise_and(lax.shift_right_logical(iv, 10), 1) * DIM)

            cp_u = pltpu.make_async_copy(ut_hbm.at[up], ug, sem_u)
            cp_i = pltpu.make_async_copy(it_hbm.at[ip], ig, sem_i)
            cp_u.start()
            cp_i.start()
            cp_u.wait()
            cp_i.wait()

            @pl.loop(0, DIM)
            def _(d):
                @pl.loop(0, WINDOW, step=LANES)
                def _(c):
                    rows = lax.iota(jnp.int32, LANES) + c
                    ucols = uh.at[pl.ds(c, LANES)][...] + d
                    icols = ih.at[pl.ds(c, LANES)][...] + d
                    uvals = plsc.load_gather(ug, [rows, ucols])
                    ivals = plsc.load_gather(ig, [rows, icols])
                    o_vmem.at[d][pl.ds(c, LANES)] = uvals * ivals

        pltpu.emit_pipeline(
            body,
            grid=(BATCH // WINDOW,),
            in_specs=[
                pl.BlockSpec((1, WINDOW), lambda i: (0, i)),
                pl.BlockSpec((1, WINDOW), lambda i: (0, i)),
            ],
            out_specs=[pl.BlockSpec((DIM, WINDOW), lambda i: (0, i))],
            core_axis_name=("core", "subcore"),
            dimension_semantics=(pltpu.PARALLEL,),
        )(u_hbm, i_hbm, o_hbm)

    out = sc_kernel(u2, i2, ut2, it2)
    return out.T


# speedup vs baseline: 16.7496x; 1.6645x over previous
import jax
import jax.numpy as jnp
from jax import lax
from jax.experimental import pallas as pl
from jax.experimental.pallas import tpu as pltpu
from jax.experimental.pallas import tpu_sc as plsc

NUM_ROWS = 1000000
NBLK = 123
NPROW = NBLK * 4096
BATCH = 16384
DIM = 64
TCC = 8192
WINDOW = 128
LANES = 16


def _tc_transpose(tT):
    def body(in_ref, out_ref):
        row = lax.broadcasted_iota(jnp.int32, (DIM, DIM), 0)
        col = lax.broadcasted_iota(jnp.int32, (DIM, DIM), 1)
        ident = jnp.where(row == col, 1.0, 0.0).astype(jnp.float32)
        t = lax.dot_general(in_ref[...], ident, (((0,), (0,)), ((), ())),
                            precision=lax.Precision.DEFAULT)
        out_ref[...] = jnp.concatenate([t[:TCC // 2], t[TCC // 2:]], axis=1)

    return pl.pallas_call(
        body,
        grid=(pl.cdiv(NUM_ROWS, TCC),),
        in_specs=[pl.BlockSpec((DIM, TCC), lambda i: (0, i))],
        out_specs=pl.BlockSpec((TCC // 2, 2 * DIM), lambda i: (i, 0)),
        out_shape=jax.ShapeDtypeStruct((NPROW, 2 * DIM), jnp.float32),
        compiler_params=pltpu.CompilerParams(
            dimension_semantics=("parallel",)),
    )(tT)


def kernel(user, item, user_table, item_table):
    ut2 = _tc_transpose(user_table.T)
    it2 = _tc_transpose(item_table.T)
    u2 = user.reshape(1, BATCH)
    i2 = item.reshape(1, BATCH)

    mesh = plsc.VectorSubcoreMesh(core_axis_name="core",
                                  subcore_axis_name="subcore")

    @pl.kernel(
        out_type=jax.ShapeDtypeStruct((DIM, BATCH), jnp.float32),
        mesh=mesh,
        compiler_params=pltpu.CompilerParams(use_tc_tiling_on_sc=False,
                                             needs_layout_passes=False),
        scratch_types=[
            pltpu.VMEM((WINDOW, 2 * DIM), jnp.float32),
            pltpu.VMEM((WINDOW, 2 * DIM), jnp.float32),
            pltpu.VMEM((WINDOW,), jnp.int32),
            pltpu.VMEM((WINDOW,), jnp.int32),
            pltpu.VMEM((WINDOW,), jnp.int32),
            pltpu.VMEM((WINDOW,), jnp.int32),
            pltpu.SemaphoreType.DMA,
            pltpu.SemaphoreType.DMA,
        ],
    )
    def sc_kernel(u_hbm, i_hbm, ut_hbm, it_hbm, o_hbm,
                  ug, ig, up, ip, uh, ih, sem_u, sem_i):
        def body(u_idx, i_idx, o_vmem):
            @pl.loop(0, WINDOW, step=LANES)
            def _(c):
                uv = u_idx.at[0][pl.ds(c, LANES)]
                iv = i_idx.at[0][pl.ds(c, LANES)]
                up.at[pl.ds(c, LANES)][...] = (
                    lax.shift_left(lax.shift_right_logical(uv, 13), 12)
                    + lax.bitwise_and(uv, 4095))
                ip.at[pl.ds(c, LANES)][...] = (
                    lax.shift_left(lax.shift_right_logical(iv, 13), 12)
                    + lax.bitwise_and(iv, 4095))
                uh.at[pl.ds(c, LANES)][...] = (
                    lax.bitwise_and(lax.shift_right_logical(uv, 12), 1) * DIM)
                ih.at[pl.ds(c, LANES)][...] = (
                    lax.bitwise_and(lax.shift_right_logical(iv, 12), 1) * DIM)

            cp_u = pltpu.make_async_copy(ut_hbm.at[up], ug, sem_u)
            cp_i = pltpu.make_async_copy(it_hbm.at[ip], ig, sem_i)
            cp_u.start()
            cp_i.start()
            cp_u.wait()
            cp_i.wait()

            @pl.loop(0, DIM)
            def _(d):
                @pl.loop(0, WINDOW, step=LANES)
                def _(c):
                    rows = lax.iota(jnp.int32, LANES) + c
                    ucols = uh.at[pl.ds(c, LANES)][...] + d
                    icols = ih.at[pl.ds(c, LANES)][...] + d
                    uvals = plsc.load_gather(ug, [rows, ucols])
                    ivals = plsc.load_gather(ig, [rows, icols])
                    o_vmem.at[d][pl.ds(c, LANES)] = uvals * ivals

        pltpu.emit_pipeline(
            body,
            grid=(BATCH // WINDOW,),
            in_specs=[
                pl.BlockSpec((1, WINDOW), lambda i: (0, i)),
                pl.BlockSpec((1, WINDOW), lambda i: (0, i)),
            ],
            out_specs=[pl.BlockSpec((DIM, WINDOW), lambda i: (0, i))],
            core_axis_name=("core", "subcore"),
            dimension_semantics=(pltpu.PARALLEL,),
        )(u_hbm, i_hbm, o_hbm)

    out = sc_kernel(u2, i2, ut2, it2)
    return out.T


# speedup vs baseline: 19.8572x; 1.1855x over previous
import jax
import jax.numpy as jnp
from jax import lax
from jax.experimental import pallas as pl
from jax.experimental.pallas import tpu as pltpu
from jax.experimental.pallas import tpu_sc as plsc

NUM_ROWS = 1000000
NBLK = 31
NPROW = NBLK * 16384
BATCH = 16384
DIM = 64
TCC = 32768
WINDOW = 128
LANES = 16


def _tc_transpose(tT):
    def body(in_ref, out_ref):
        row = lax.broadcasted_iota(jnp.int32, (DIM, DIM), 0)
        col = lax.broadcasted_iota(jnp.int32, (DIM, DIM), 1)
        ident = jnp.where(row == col, 1.0, 0.0).astype(jnp.float32)
        t = lax.dot_general(in_ref[...], ident, (((0,), (0,)), ((), ())),
                            precision=lax.Precision.DEFAULT)
        out_ref[...] = jnp.concatenate([t[:TCC // 2], t[TCC // 2:]], axis=1)

    return pl.pallas_call(
        body,
        grid=(pl.cdiv(NUM_ROWS, TCC),),
        in_specs=[pl.BlockSpec((DIM, TCC), lambda i: (0, i))],
        out_specs=pl.BlockSpec((TCC // 2, 2 * DIM), lambda i: (i, 0)),
        out_shape=jax.ShapeDtypeStruct((NPROW, 2 * DIM), jnp.float32),
        compiler_params=pltpu.CompilerParams(
            dimension_semantics=("parallel",)),
    )(tT)


def kernel(user, item, user_table, item_table):
    ut2 = _tc_transpose(user_table.T)
    it2 = _tc_transpose(item_table.T)
    u2 = user.reshape(1, BATCH)
    i2 = item.reshape(1, BATCH)

    mesh = plsc.VectorSubcoreMesh(core_axis_name="core",
                                  subcore_axis_name="subcore")

    @pl.kernel(
        out_type=jax.ShapeDtypeStruct((DIM, BATCH), jnp.float32),
        mesh=mesh,
        compiler_params=pltpu.CompilerParams(use_tc_tiling_on_sc=False,
                                             needs_layout_passes=False),
        scratch_types=[
            pltpu.VMEM((WINDOW, 2 * DIM), jnp.float32),
            pltpu.VMEM((WINDOW, 2 * DIM), jnp.float32),
            pltpu.VMEM((WINDOW,), jnp.int32),
            pltpu.VMEM((WINDOW,), jnp.int32),
            pltpu.VMEM((WINDOW,), jnp.int32),
            pltpu.VMEM((WINDOW,), jnp.int32),
            pltpu.SemaphoreType.DMA,
            pltpu.SemaphoreType.DMA,
        ],
    )
    def sc_kernel(u_hbm, i_hbm, ut_hbm, it_hbm, o_hbm,
                  ug, ig, up, ip, uh, ih, sem_u, sem_i):
        def body(u_idx, i_idx, o_vmem):
            @pl.loop(0, WINDOW, step=LANES)
            def _(c):
                uv = u_idx.at[0][pl.ds(c, LANES)]
                iv = i_idx.at[0][pl.ds(c, LANES)]
                up.at[pl.ds(c, LANES)][...] = (
                    lax.shift_left(lax.shift_right_logical(uv, 15), 14)
                    + lax.bitwise_and(uv, 16383))
                ip.at[pl.ds(c, LANES)][...] = (
                    lax.shift_left(lax.shift_right_logical(iv, 15), 14)
                    + lax.bitwise_and(iv, 16383))
                uh.at[pl.ds(c, LANES)][...] = (
                    lax.bitwise_and(lax.shift_right_logical(uv, 14), 1) * DIM)
                ih.at[pl.ds(c, LANES)][...] = (
                    lax.bitwise_and(lax.shift_right_logical(iv, 14), 1) * DIM)

            cp_u = pltpu.make_async_copy(ut_hbm.at[up], ug, sem_u)
            cp_i = pltpu.make_async_copy(it_hbm.at[ip], ig, sem_i)
            cp_u.start()
            cp_i.start()
            cp_u.wait()
            cp_i.wait()

            @pl.loop(0, DIM)
            def _(d):
                @pl.loop(0, WINDOW, step=LANES)
                def _(c):
                    rows = lax.iota(jnp.int32, LANES) + c
                    ucols = uh.at[pl.ds(c, LANES)][...] + d
                    icols = ih.at[pl.ds(c, LANES)][...] + d
                    uvals = plsc.load_gather(ug, [rows, ucols])
                    ivals = plsc.load_gather(ig, [rows, icols])
                    o_vmem.at[d][pl.ds(c, LANES)] = uvals * ivals

        pltpu.emit_pipeline(
            body,
            grid=(BATCH // WINDOW,),
            in_specs=[
                pl.BlockSpec((1, WINDOW), lambda i: (0, i)),
                pl.BlockSpec((1, WINDOW), lambda i: (0, i)),
            ],
            out_specs=[pl.BlockSpec((DIM, WINDOW), lambda i: (0, i))],
            core_axis_name=("core", "subcore"),
            dimension_semantics=(pltpu.PARALLEL,),
        )(u_hbm, i_hbm, o_hbm)

    out = sc_kernel(u2, i2, ut2, it2)
    return out.T


# speedup vs baseline: 20.5969x; 1.0373x over previous
import jax
import jax.numpy as jnp
from jax import lax
from jax.experimental import pallas as pl
from jax.experimental.pallas import tpu as pltpu
from jax.experimental.pallas import tpu_sc as plsc

NUM_ROWS = 1000000
NBLK = 31
NPROW = NBLK * 16384
BATCH = 16384
DIM = 64
TCC = 32768
WINDOW = 128
LANES = 16

_MESH = plsc.VectorSubcoreMesh(core_axis_name="core",
                               subcore_axis_name="subcore")
_SC_PARAMS = pltpu.CompilerParams(use_tc_tiling_on_sc=False,
                                  needs_layout_passes=False)


def _sc_scratch():
    return [
        pltpu.VMEM((WINDOW, 2 * DIM), jnp.float32),
        pltpu.VMEM((WINDOW,), jnp.int32),
        pltpu.VMEM((WINDOW,), jnp.int32),
        pltpu.SemaphoreType.DMA,
    ]


def _tc_transpose(tT):
    def body(in_ref, out_ref):
        row = lax.broadcasted_iota(jnp.int32, (DIM, DIM), 0)
        col = lax.broadcasted_iota(jnp.int32, (DIM, DIM), 1)
        ident = jnp.where(row == col, 1.0, 0.0).astype(jnp.float32)
        t = lax.dot_general(in_ref[...], ident, (((0,), (0,)), ((), ())),
                            precision=lax.Precision.DEFAULT)
        out_ref[...] = jnp.concatenate([t[:TCC // 2], t[TCC // 2:]], axis=1)

    return pl.pallas_call(
        body,
        grid=(pl.cdiv(NUM_ROWS, TCC),),
        in_specs=[pl.BlockSpec((DIM, TCC), lambda i: (0, i))],
        out_specs=pl.BlockSpec((TCC // 2, 2 * DIM), lambda i: (i, 0)),
        out_shape=jax.ShapeDtypeStruct((NPROW, 2 * DIM), jnp.float32),
        compiler_params=pltpu.CompilerParams(
            dimension_semantics=("parallel",)),
    )(tT)


def _idx_split(idx_vec, pid, hoff, c):
    v = idx_vec.at[0][pl.ds(c, LANES)]
    pid.at[pl.ds(c, LANES)][...] = (
        lax.shift_left(lax.shift_right_logical(v, 15), 14)
        + lax.bitwise_and(v, 16383))
    hoff.at[pl.ds(c, LANES)][...] = (
        lax.bitwise_and(lax.shift_right_logical(v, 14), 1) * DIM)


def _sc_gather_u(idx2, tab2):
    @pl.kernel(
        out_type=jax.ShapeDtypeStruct((DIM, BATCH), jnp.float32),
        mesh=_MESH,
        compiler_params=_SC_PARAMS,
        scratch_types=_sc_scratch(),
    )
    def sc_kernel(u_hbm, t_hbm, o_hbm, gbuf, pid, hoff, sem):
        def body(u_idx, o_vmem):
            @pl.loop(0, WINDOW, step=LANES)
            def _(c):
                _idx_split(u_idx, pid, hoff, c)

            cp = pltpu.make_async_copy(t_hbm.at[pid], gbuf, sem)
            cp.start()
            cp.wait()

            @pl.loop(0, DIM)
            def _(d):
                @pl.loop(0, WINDOW, step=LANES)
                def _(c):
                    rows = lax.iota(jnp.int32, LANES) + c
                    cols = hoff.at[pl.ds(c, LANES)][...] + d
                    o_vmem.at[d][pl.ds(c, LANES)] = plsc.load_gather(
                        gbuf, [rows, cols])

        pltpu.emit_pipeline(
            body,
            grid=(BATCH // WINDOW,),
            in_specs=[pl.BlockSpec((1, WINDOW), lambda i: (0, i))],
            out_specs=[pl.BlockSpec((DIM, WINDOW), lambda i: (0, i))],
            core_axis_name=("core", "subcore"),
            dimension_semantics=(pltpu.PARALLEL,),
        )(u_hbm, o_hbm)

    return sc_kernel(idx2, tab2)


def _sc_gather_i_mul(idx2, tab2, gu):
    @pl.kernel(
        out_type=jax.ShapeDtypeStruct((DIM, BATCH), jnp.float32),
        mesh=_MESH,
        compiler_params=_SC_PARAMS,
        scratch_types=_sc_scratch(),
    )
    def sc_kernel(i_hbm, t_hbm, g_hbm, o_hbm, gbuf, pid, hoff, sem):
        def body(i_idx, g_blk, o_vmem):
            @pl.loop(0, WINDOW, step=LANES)
            def _(c):
                _idx_split(i_idx, pid, hoff, c)

            cp = pltpu.make_async_copy(t_hbm.at[pid], gbuf, sem)
            cp.start()
            cp.wait()

            @pl.loop(0, DIM)
            def _(d):
                @pl.loop(0, WINDOW, step=LANES)
                def _(c):
                    rows = lax.iota(jnp.int32, LANES) + c
                    cols = hoff.at[pl.ds(c, LANES)][...] + d
                    ivals = plsc.load_gather(gbuf, [rows, cols])
                    o_vmem.at[d][pl.ds(c, LANES)] = (
                        ivals * g_blk.at[d][pl.ds(c, LANES)])

        pltpu.emit_pipeline(
            body,
            grid=(BATCH // WINDOW,),
            in_specs=[
                pl.BlockSpec((1, WINDOW), lambda i: (0, i)),
                pl.BlockSpec((DIM, WINDOW), lambda i: (0, i)),
            ],
            out_specs=[pl.BlockSpec((DIM, WINDOW), lambda i: (0, i))],
            core_axis_name=("core", "subcore"),
            dimension_semantics=(pltpu.PARALLEL,),
        )(i_hbm, g_hbm, o_hbm)

    return sc_kernel(idx2, tab2, gu)


def kernel(user, item, user_table, item_table):
    ut2 = _tc_transpose(user_table.T)
    it2 = _tc_transpose(item_table.T)
    u2 = user.reshape(1, BATCH)
    i2 = item.reshape(1, BATCH)

    gu = _sc_gather_u(u2, ut2)
    out = _sc_gather_i_mul(i2, it2, gu)
    return out.T
